# baseline (speedup 1.0000x reference)
import functools

import jax
import jax.numpy as jnp
from jax.experimental import pallas as pl

BATCH = 16384
FIELD_DIM = 1000
EMBED_DIM = 64
BLOCK_M = 1024


def _embed_block(idx_ref, w_ref, out_ref):
    mask = (idx_ref[...] != 0).astype(jnp.float32)
    out_ref[...] = jnp.dot(mask, w_ref[...],
                           preferred_element_type=jnp.float32)


@functools.partial(jax.jit, static_argnames=())
def kernel(indices, W):
    grid = (BATCH // BLOCK_M,)
    return pl.pallas_call(
        _embed_block,
        grid=grid,
        in_specs=[
            pl.BlockSpec((BLOCK_M, FIELD_DIM), lambda i: (i, 0)),
            pl.BlockSpec((FIELD_DIM, EMBED_DIM), lambda i: (0, 0)),
        ],
        out_specs=pl.BlockSpec((BLOCK_M, EMBED_DIM), lambda i: (i, 0)),
        out_shape=jax.ShapeDtypeStruct((BATCH, EMBED_DIM), jnp.float32),
    )(indices, W)
